# baseline (speedup 1.0000x reference)
import functools

import jax
import jax.numpy as jnp
from jax import lax
from jax.experimental import pallas as pl
from jax.experimental.pallas import tpu as pltpu
from jax.experimental.pallas import tpu_sc as plsc

_NUM_CORES = 2
_NUM_SUBCORES = 16
_NUM_WORKERS = _NUM_CORES * _NUM_SUBCORES
_CHUNK = 512


@functools.partial(jax.jit, static_argnames=("n", "dim"))
def _gather_flat(idx, table, n, dim):
    per_worker = n // _NUM_WORKERS
    steps = per_worker // _CHUNK
    mesh = plsc.VectorSubcoreMesh(core_axis_name="c", subcore_axis_name="s")

    @functools.partial(
        pl.kernel,
        out_type=jax.ShapeDtypeStruct((n, dim), jnp.float32),
        mesh=mesh,
        scratch_types=[
            pltpu.VMEM((_CHUNK,), jnp.int32),
            pltpu.VMEM((_CHUNK, dim), jnp.float32),
            pltpu.SemaphoreType.DMA,
        ],
        compiler_params=pltpu.CompilerParams(use_tc_tiling_on_sc=False),
    )
    def body(idx_hbm, table_hbm, out_hbm, idx_v, rows_v, sem):
        wid = lax.axis_index("s") * _NUM_CORES + lax.axis_index("c")
        base = wid * per_worker

        def step(i, carry):
            off = base + i * _CHUNK
            pltpu.sync_copy(idx_hbm.at[pl.ds(off, _CHUNK)], idx_v)
            pltpu.async_copy(table_hbm.at[idx_v], rows_v, sem).wait()
            pltpu.sync_copy(rows_v, out_hbm.at[pl.ds(off, _CHUNK)])
            return carry

        lax.fori_loop(0, steps, step, 0)

    return body(idx, table)


def kernel(tokens, embedding_table):
    batch, seq = tokens.shape
    _, dim = embedding_table.shape
    n = batch * seq
    idx = tokens.reshape(n).astype(jnp.int32)
    out = _gather_flat(idx, embedding_table, n, dim)
    return out.reshape(batch, seq, dim)


# speedup vs baseline: 1.0774x; 1.0774x over previous
import functools

import jax
import jax.numpy as jnp
from jax import lax
from jax.experimental import pallas as pl
from jax.experimental.pallas import tpu as pltpu
from jax.experimental.pallas import tpu_sc as plsc

_NUM_CORES = 2
_NUM_SUBCORES = 16
_NUM_WORKERS = _NUM_CORES * _NUM_SUBCORES
_CHUNK = 512


@functools.partial(jax.jit, static_argnames=("n", "dim"))
def _gather_flat(idx, table, n, dim):
    per_worker = n // _NUM_WORKERS
    steps = per_worker // _CHUNK
    assert steps % 2 == 0 and steps >= 4
    mesh = plsc.VectorSubcoreMesh(core_axis_name="c", subcore_axis_name="s")

    @functools.partial(
        pl.kernel,
        out_type=jax.ShapeDtypeStruct((n, dim), jnp.float32),
        mesh=mesh,
        scratch_types=[
            pltpu.VMEM((_CHUNK,), jnp.int32),
            pltpu.VMEM((_CHUNK,), jnp.int32),
            pltpu.VMEM((_CHUNK, dim), jnp.float32),
            pltpu.VMEM((_CHUNK, dim), jnp.float32),
            pltpu.SemaphoreType.DMA,
            pltpu.SemaphoreType.DMA,
            pltpu.SemaphoreType.DMA,
            pltpu.SemaphoreType.DMA,
            pltpu.SemaphoreType.DMA,
            pltpu.SemaphoreType.DMA,
        ],
        compiler_params=pltpu.CompilerParams(use_tc_tiling_on_sc=False),
    )
    def body(idx_hbm, table_hbm, out_hbm, idx0, idx1, rows0, rows1,
             i0, i1, g0, g1, s0, s1):
        wid = lax.axis_index("s") * _NUM_CORES + lax.axis_index("c")
        base = wid * per_worker
        idx_b = (idx0, idx1)
        rows_b = (rows0, rows1)
        i_sem = (i0, i1)
        g_sem = (g0, g1)
        s_sem = (s0, s1)

        def fire_idx(b, chunk):
            pltpu.async_copy(
                idx_hbm.at[pl.ds(base + chunk * _CHUNK, _CHUNK)],
                idx_b[b], i_sem[b])

        def wait_idx(b):
            pltpu.make_async_copy(
                idx_hbm.at[pl.ds(base, _CHUNK)], idx_b[b], i_sem[b]).wait()

        def fire_gather(b):
            pltpu.async_copy(table_hbm.at[idx_b[b]], rows_b[b], g_sem[b])

        def wait_gather(b):
            pltpu.make_async_copy(
                table_hbm.at[idx_b[b]], rows_b[b], g_sem[b]).wait()

        def fire_store(b, chunk):
            pltpu.async_copy(
                rows_b[b],
                out_hbm.at[pl.ds(base + chunk * _CHUNK, _CHUNK)], s_sem[b])

        def wait_store(b):
            pltpu.make_async_copy(
                rows_b[b], out_hbm.at[pl.ds(base, _CHUNK)], s_sem[b]).wait()

        fire_idx(0, 0)
        fire_idx(1, 1)
        wait_idx(0)
        fire_gather(0)
        wait_idx(1)
        fire_gather(1)
        wait_gather(0)
        fire_store(0, 0)
        fire_idx(0, 2)

        def outer(g, carry):
            c0 = 2 * g
            wait_idx(0)
            wait_store(0)
            fire_gather(0)
            wait_gather(1)
            fire_store(1, c0 - 1)
            fire_idx(1, c0 + 1)
            wait_idx(1)
            wait_store(1)
            fire_gather(1)
            wait_gather(0)
            fire_store(0, c0)
            fire_idx(0, jnp.minimum(c0 + 2, steps - 1))
            return carry

        lax.fori_loop(1, steps // 2, outer, 0)

        wait_gather(1)
        fire_store(1, steps - 1)
        wait_idx(0)
        wait_store(0)
        wait_store(1)

    return body(idx, table)


def kernel(tokens, embedding_table):
    batch, seq = tokens.shape
    _, dim = embedding_table.shape
    n = batch * seq
    idx = tokens.reshape(n).astype(jnp.int32)
    out = _gather_flat(idx, embedding_table, n, dim)
    return out.reshape(batch, seq, dim)


# speedup vs baseline: 1.0776x; 1.0002x over previous
import functools

import jax
import jax.numpy as jnp
from jax import lax
from jax.experimental import pallas as pl
from jax.experimental.pallas import tpu as pltpu
from jax.experimental.pallas import tpu_sc as plsc

_NUM_CORES = 2
_NUM_SUBCORES = 16
_NUM_WORKERS = _NUM_CORES * _NUM_SUBCORES
_GROUP = 4


@functools.partial(jax.jit, static_argnames=("batch", "seq", "dim"))
def _embed(tokens, table, batch, seq, dim):
    rows_per_worker = batch // _NUM_WORKERS
    steps = rows_per_worker // _GROUP
    assert steps % 2 == 0 and steps >= 4
    mesh = plsc.VectorSubcoreMesh(core_axis_name="c", subcore_axis_name="s")

    @functools.partial(
        pl.kernel,
        out_type=jax.ShapeDtypeStruct((batch, seq, dim), jnp.float32),
        mesh=mesh,
        scratch_types=[
            pltpu.VMEM((_GROUP, seq), jnp.int32),
            pltpu.VMEM((_GROUP, seq), jnp.int32),
            pltpu.VMEM((_GROUP, seq, dim), jnp.float32),
            pltpu.VMEM((_GROUP, seq, dim), jnp.float32),
            pltpu.SemaphoreType.DMA,
            pltpu.SemaphoreType.DMA,
            pltpu.SemaphoreType.DMA,
            pltpu.SemaphoreType.DMA,
            pltpu.SemaphoreType.DMA,
            pltpu.SemaphoreType.DMA,
        ],
        compiler_params=pltpu.CompilerParams(use_tc_tiling_on_sc=False),
    )
    def body(tok_hbm, table_hbm, out_hbm, idx0, idx1, rows0, rows1,
             i0, i1, g0, g1, s0, s1):
        wid = lax.axis_index("s") * _NUM_CORES + lax.axis_index("c")
        base = wid * rows_per_worker
        idx_b = (idx0, idx1)
        rows_b = (rows0, rows1)
        i_sem = (i0, i1)
        g_sem = (g0, g1)
        s_sem = (s0, s1)

        def fire_idx(b, grp):
            pltpu.async_copy(
                tok_hbm.at[pl.ds(base + grp * _GROUP, _GROUP)],
                idx_b[b], i_sem[b])

        def wait_idx(b):
            pltpu.make_async_copy(
                tok_hbm.at[pl.ds(base, _GROUP)], idx_b[b], i_sem[b]).wait()

        def fire_gather(b):
            for j in range(_GROUP):
                pltpu.async_copy(
                    table_hbm.at[idx_b[b].at[j]], rows_b[b].at[j], g_sem[b])

        def wait_gather(b):
            for j in range(_GROUP):
                pltpu.make_async_copy(
                    table_hbm.at[idx_b[b].at[j]], rows_b[b].at[j],
                    g_sem[b]).wait()

        def fire_store(b, grp):
            pltpu.async_copy(
                rows_b[b],
                out_hbm.at[pl.ds(base + grp * _GROUP, _GROUP)], s_sem[b])

        def wait_store(b):
            pltpu.make_async_copy(
                rows_b[b], out_hbm.at[pl.ds(base, _GROUP)], s_sem[b]).wait()

        fire_idx(0, 0)
        fire_idx(1, 1)
        wait_idx(0)
        fire_gather(0)
        wait_idx(1)
        fire_gather(1)
        wait_gather(0)
        fire_store(0, 0)
        fire_idx(0, 2)

        def outer(g, carry):
            c0 = 2 * g
            wait_idx(0)
            wait_store(0)
            fire_gather(0)
            wait_gather(1)
            fire_store(1, c0 - 1)
            fire_idx(1, c0 + 1)
            wait_idx(1)
            wait_store(1)
            fire_gather(1)
            wait_gather(0)
            fire_store(0, c0)
            fire_idx(0, jnp.minimum(c0 + 2, steps - 1))
            return carry

        lax.fori_loop(1, steps // 2, outer, 0)

        wait_gather(1)
        fire_store(1, steps - 1)
        wait_idx(0)
        wait_store(0)
        wait_store(1)

    return body(tokens, table)


def kernel(tokens, embedding_table):
    batch, seq = tokens.shape
    _, dim = embedding_table.shape
    return _embed(tokens.astype(jnp.int32), embedding_table, batch, seq, dim)


# speedup vs baseline: 1.7735x; 1.6458x over previous
import functools

import jax
import jax.numpy as jnp
from jax import lax
from jax.experimental import pallas as pl
from jax.experimental.pallas import tpu as pltpu
from jax.experimental.pallas import tpu_sc as plsc

_NUM_CORES = 2
_NUM_SUBCORES = 16
_NUM_WORKERS = _NUM_CORES * _NUM_SUBCORES
_GROUP = 4


@functools.partial(jax.jit, static_argnames=("batch", "seq", "dim"))
def _embed(tokens, table, batch, seq, dim):
    rows_per_worker = batch // _NUM_WORKERS
    steps = rows_per_worker // _GROUP
    assert steps % 2 == 0 and steps >= 4
    mesh = plsc.VectorSubcoreMesh(core_axis_name="c", subcore_axis_name="s")

    @functools.partial(
        pl.kernel,
        out_type=jax.ShapeDtypeStruct((batch, seq, 2 * dim), jnp.float32),
        mesh=mesh,
        scratch_types=[
            pltpu.VMEM((_GROUP, seq), jnp.int32),
            pltpu.VMEM((_GROUP, seq), jnp.int32),
            pltpu.VMEM((_GROUP, seq, dim), jnp.float32),
            pltpu.VMEM((_GROUP, seq, dim), jnp.float32),
            pltpu.SemaphoreType.DMA,
            pltpu.SemaphoreType.DMA,
            pltpu.SemaphoreType.DMA,
            pltpu.SemaphoreType.DMA,
            pltpu.SemaphoreType.DMA,
            pltpu.SemaphoreType.DMA,
        ],
        compiler_params=pltpu.CompilerParams(use_tc_tiling_on_sc=False),
    )
    def body(tok_hbm, table_hbm, out_hbm, idx0, idx1, rows0, rows1,
             i0, i1, g0, g1, s0, s1):
        wid = lax.axis_index("s") * _NUM_CORES + lax.axis_index("c")
        base = wid * rows_per_worker
        idx_b = (idx0, idx1)
        rows_b = (rows0, rows1)
        i_sem = (i0, i1)
        g_sem = (g0, g1)
        s_sem = (s0, s1)

        def fire_idx(b, grp):
            pltpu.async_copy(
                tok_hbm.at[pl.ds(base + grp * _GROUP, _GROUP)],
                idx_b[b], i_sem[b])

        def wait_idx(b):
            pltpu.make_async_copy(
                tok_hbm.at[pl.ds(base, _GROUP)], idx_b[b], i_sem[b]).wait()

        def fire_gather(b):
            for j in range(_GROUP):
                pltpu.async_copy(
                    table_hbm.at[idx_b[b].at[j]], rows_b[b].at[j], g_sem[b])

        def wait_gather(b):
            for j in range(_GROUP):
                pltpu.make_async_copy(
                    table_hbm.at[idx_b[b].at[j]], rows_b[b].at[j],
                    g_sem[b]).wait()

        def fire_store(b, grp):
            pltpu.async_copy(
                rows_b[b],
                out_hbm.at[pl.ds(base + grp * _GROUP, _GROUP), :,
                           pl.ds(0, dim)], s_sem[b])

        def wait_store(b):
            pltpu.make_async_copy(
                rows_b[b],
                out_hbm.at[pl.ds(base, _GROUP), :, pl.ds(0, dim)],
                s_sem[b]).wait()

        fire_idx(0, 0)
        fire_idx(1, 1)
        wait_idx(0)
        fire_gather(0)
        wait_idx(1)
        fire_gather(1)
        wait_gather(0)
        fire_store(0, 0)
        fire_idx(0, 2)

        def outer(g, carry):
            c0 = 2 * g
            wait_idx(0)
            wait_store(0)
            fire_gather(0)
            wait_gather(1)
            fire_store(1, c0 - 1)
            fire_idx(1, c0 + 1)
            wait_idx(1)
            wait_store(1)
            fire_gather(1)
            wait_gather(0)
            fire_store(0, c0)
            fire_idx(0, jnp.minimum(c0 + 2, steps - 1))
            return carry

        lax.fori_loop(1, steps // 2, outer, 0)

        wait_gather(1)
        fire_store(1, steps - 1)
        wait_idx(0)
        wait_store(0)
        wait_store(1)

    return body(tokens, table)[:, :, :dim]


def kernel(tokens, embedding_table):
    batch, seq = tokens.shape
    _, dim = embedding_table.shape
    return _embed(tokens.astype(jnp.int32), embedding_table, batch, seq, dim)
